# baseline (speedup 1.0000x reference)
import jax
import jax.numpy as jnp
from jax import lax
from jax.experimental import pallas as pl
from jax.experimental.pallas import tpu as pltpu

EPS = 1e-12


def _dotp(a, b):
    return lax.dot_general(a, b, (((1,), (0,)), ((), ())),
                           preferred_element_type=jnp.float32,
                           precision=lax.Precision.HIGHEST)


def _ctx_body(h_ref, s_ref, b_ref, o_ref):
    bm = b_ref.shape[0]
    ctx = jnp.zeros((bm, 1), jnp.int32)
    for c in range(4):
        pc = _dotp(h_ref[:, c, :], s_ref[:])
        ctx = ctx + jnp.where(pc > b_ref[:, c:c + 1], 1 << c, 0)
    o_ref[:] = ctx


def _ctx_ids(H, s, B):
    N = H.shape[0]
    bm = 128
    return pl.pallas_call(
        _ctx_body,
        grid=(N // bm,),
        in_specs=[pl.BlockSpec((bm, 4, 4096), lambda i: (i, 0, 0)),
                  pl.BlockSpec(s.shape, lambda i: (0, 0)),
                  pl.BlockSpec((bm, 4), lambda i: (i, 0))],
        out_specs=pl.BlockSpec((bm, 1), lambda i: (i, 0)),
        out_shape=jax.ShapeDtypeStruct((N, 1), jnp.int32),
    )(H, s, B)


def _rev_sigmoid(p):
    pc = jnp.clip(p, EPS, 1.0 - EPS)
    return jnp.log(pc) - jnp.log1p(-pc)


def _row_dma(ids_ref, w_hbm, wg_v, sem, n):
    return pltpu.make_async_copy(w_hbm.at[n, ids_ref[n]], wg_v.at[n], sem)


def _mix_body(ids0_ref, ids1_ref, x_ref, s_ref, w0_hbm, w1_hbm, w2_ref,
              h2_ref, b2_ref, o_ref, wg0_v, wg1_v, sem0, sem1):
    for ids_ref, w_hbm, wg_v, sem in ((ids0_ref, w0_hbm, wg0_v, sem0),
                                      (ids1_ref, w1_hbm, wg1_v, sem1)):
        lax.fori_loop(
            0, 1024,
            lambda n, _, a=(ids_ref, w_hbm, wg_v, sem):
                (_row_dma(*a, n).start(), 0)[1], 0)
    for ids_ref, w_hbm, wg_v, sem in ((ids0_ref, w0_hbm, wg0_v, sem0),
                                      (ids1_ref, w1_hbm, wg1_v, sem1)):
        lax.fori_loop(
            0, 1024,
            lambda n, _, a=(ids_ref, w_hbm, wg_v, sem):
                (_row_dma(*a, n).wait(), 0)[1], 0)

    bias = jax.nn.sigmoid(jnp.ones((1, 1), jnp.float32))
    x0 = _rev_sigmoid(jnp.concatenate([x_ref[:], bias], axis=0))
    p0 = jax.nn.sigmoid(_dotp(wg0_v[:], x0))
    x1 = _rev_sigmoid(jnp.concatenate([p0, bias], axis=0))
    p1 = jax.nn.sigmoid(_dotp(wg1_v[:], x1))
    x2 = _rev_sigmoid(jnp.concatenate([p1, bias], axis=0))
    l2a = _dotp(w2_ref[:], x2)
    pr2 = _dotp(h2_ref[:], s_ref[:])
    pw = 1 << lax.broadcasted_iota(jnp.int32, (4, 1), 0)
    c2 = jnp.sum(jnp.where(pr2 > b2_ref[:], pw, 0), keepdims=True)
    oh = lax.broadcasted_iota(jnp.int32, (16, 1), 0) == c2
    p2 = jax.nn.sigmoid(jnp.sum(jnp.where(oh, l2a, 0.0), keepdims=True))
    o_ref[:] = jnp.concatenate([p0, p1, p2], axis=0)


def _mix(ids0, ids1, x, s, W0, W1, W2f, H2f, B2t):
    grid_spec = pltpu.PrefetchScalarGridSpec(
        num_scalar_prefetch=2,
        grid=(1,),
        in_specs=[
            pl.BlockSpec(x.shape, lambda i, *_: (0, 0)),
            pl.BlockSpec(s.shape, lambda i, *_: (0, 0)),
            pl.BlockSpec(memory_space=pltpu.MemorySpace.HBM),
            pl.BlockSpec(memory_space=pltpu.MemorySpace.HBM),
            pl.BlockSpec(W2f.shape, lambda i, *_: (0, 0)),
            pl.BlockSpec(H2f.shape, lambda i, *_: (0, 0)),
            pl.BlockSpec(B2t.shape, lambda i, *_: (0, 0)),
        ],
        out_specs=pl.BlockSpec((2049, 1), lambda i, *_: (0, 0)),
        scratch_shapes=[pltpu.VMEM((1024, 1025), jnp.float32),
                        pltpu.VMEM((1024, 1025), jnp.float32),
                        pltpu.SemaphoreType.DMA,
                        pltpu.SemaphoreType.DMA],
    )
    return pl.pallas_call(
        _mix_body,
        grid_spec=grid_spec,
        out_shape=jax.ShapeDtypeStruct((2049, 1), jnp.float32),
    )(ids0, ids1, x, s, W0, W1, W2f, H2f, B2t)


def kernel(inputs, side_info, W0, W1, W2, H0, H1, H2, B0, B1, B2):
    s = side_info.reshape(4096, 1)
    ids0 = _ctx_ids(H0, s, B0).reshape(1024)
    ids1 = _ctx_ids(H1, s, B1).reshape(1024)
    out = _mix(ids0, ids1, inputs.reshape(1024, 1), s, W0, W1,
               W2.reshape(16, 1025), H2.reshape(4, 4096), B2.reshape(4, 1))
    return out.reshape(2049)


# speedup vs baseline: 1.0586x; 1.0586x over previous
import jax
import jax.numpy as jnp
from jax import lax
from jax.experimental import pallas as pl
from jax.experimental.pallas import tpu as pltpu

EPS = 1e-12


def _dotp(a, b):
    return lax.dot_general(a, b, (((1,), (0,)), ((), ())),
                           preferred_element_type=jnp.float32,
                           precision=lax.Precision.HIGHEST)


def _ctx_body(h_ref, s_ref, b_ref, o_ref):
    bm = b_ref.shape[0]
    ctx = jnp.zeros((bm, 1), jnp.int32)
    for c in range(4):
        pc = _dotp(h_ref[:, c, :], s_ref[:])
        ctx = ctx + jnp.where(pc > b_ref[:, c:c + 1], 1 << c, 0)
    o_ref[:] = ctx


def _ctx_ids(H, s, B):
    N = H.shape[0]
    bm = 128
    return pl.pallas_call(
        _ctx_body,
        grid=(N // bm,),
        in_specs=[pl.BlockSpec((bm, 4, 4096), lambda i: (i, 0, 0)),
                  pl.BlockSpec(s.shape, lambda i: (0, 0)),
                  pl.BlockSpec((bm, 4), lambda i: (i, 0))],
        out_specs=pl.BlockSpec((bm, 1), lambda i: (i, 0)),
        out_shape=jax.ShapeDtypeStruct((N, 1), jnp.int32),
    )(H, s, B)


def _rev_sigmoid(p):
    pc = jnp.clip(p, EPS, 1.0 - EPS)
    return jnp.log(pc) - jnp.log1p(-pc)


def _row_dma(ids_ref, w_hbm, wg_v, sem, n):
    return pltpu.make_async_copy(w_hbm.at[n, ids_ref[n]], wg_v.at[n], sem)


def _mix_body(ids0_ref, ids1_ref, x_ref, s_ref, w0_hbm, w1_hbm, w2_ref,
              h2_ref, b2_ref, o_ref, wg0_v, wg1_v, sem0, sem1):
    for ids_ref, w_hbm, wg_v, sem in ((ids0_ref, w0_hbm, wg0_v, sem0),
                                      (ids1_ref, w1_hbm, wg1_v, sem1)):
        def _start8(i, _, a=(ids_ref, w_hbm, wg_v, sem)):
            for u in range(8):
                _row_dma(*a, i * 8 + u).start()
            return 0
        lax.fori_loop(0, 128, _start8, 0)
    pltpu.make_async_copy(w0_hbm.at[:, 0], wg0_v, sem0).wait()
    pltpu.make_async_copy(w1_hbm.at[:, 0], wg1_v, sem1).wait()

    bias = jax.nn.sigmoid(jnp.ones((1, 1), jnp.float32))
    x0 = _rev_sigmoid(jnp.concatenate([x_ref[:], bias], axis=0))
    p0 = jax.nn.sigmoid(_dotp(wg0_v[:], x0))
    x1 = _rev_sigmoid(jnp.concatenate([p0, bias], axis=0))
    p1 = jax.nn.sigmoid(_dotp(wg1_v[:], x1))
    x2 = _rev_sigmoid(jnp.concatenate([p1, bias], axis=0))
    l2a = _dotp(w2_ref[:], x2)
    pr2 = _dotp(h2_ref[:], s_ref[:])
    pw = 1 << lax.broadcasted_iota(jnp.int32, (4, 1), 0)
    c2 = jnp.sum(jnp.where(pr2 > b2_ref[:], pw, 0), keepdims=True)
    oh = lax.broadcasted_iota(jnp.int32, (16, 1), 0) == c2
    p2 = jax.nn.sigmoid(jnp.sum(jnp.where(oh, l2a, 0.0), keepdims=True))
    o_ref[:] = jnp.concatenate([p0, p1, p2], axis=0)


def _mix(ids0, ids1, x, s, W0, W1, W2f, H2f, B2t):
    grid_spec = pltpu.PrefetchScalarGridSpec(
        num_scalar_prefetch=2,
        grid=(1,),
        in_specs=[
            pl.BlockSpec(x.shape, lambda i, *_: (0, 0)),
            pl.BlockSpec(s.shape, lambda i, *_: (0, 0)),
            pl.BlockSpec(memory_space=pltpu.MemorySpace.HBM),
            pl.BlockSpec(memory_space=pltpu.MemorySpace.HBM),
            pl.BlockSpec(W2f.shape, lambda i, *_: (0, 0)),
            pl.BlockSpec(H2f.shape, lambda i, *_: (0, 0)),
            pl.BlockSpec(B2t.shape, lambda i, *_: (0, 0)),
        ],
        out_specs=pl.BlockSpec((2049, 1), lambda i, *_: (0, 0)),
        scratch_shapes=[pltpu.VMEM((1024, 1025), jnp.float32),
                        pltpu.VMEM((1024, 1025), jnp.float32),
                        pltpu.SemaphoreType.DMA,
                        pltpu.SemaphoreType.DMA],
    )
    return pl.pallas_call(
        _mix_body,
        grid_spec=grid_spec,
        out_shape=jax.ShapeDtypeStruct((2049, 1), jnp.float32),
    )(ids0, ids1, x, s, W0, W1, W2f, H2f, B2t)


def kernel(inputs, side_info, W0, W1, W2, H0, H1, H2, B0, B1, B2):
    s = side_info.reshape(4096, 1)
    ids0 = _ctx_ids(H0, s, B0).reshape(1024)
    ids1 = _ctx_ids(H1, s, B1).reshape(1024)
    out = _mix(ids0, ids1, inputs.reshape(1024, 1), s, W0, W1,
               W2.reshape(16, 1025), H2.reshape(4, 4096), B2.reshape(4, 1))
    return out.reshape(2049)
